# baseline (speedup 1.0000x reference)
import functools

import jax
import jax.numpy as jnp
from jax import lax
from jax.experimental import pallas as pl
from jax.experimental.pallas import tpu as pltpu
from jax.experimental.pallas import tpu_sc as plsc

N = 1024
NPAIR = 64
CW = 128
KA = 8
P = 4
KTC = 56
SCP = NPAIR - KTC
L = 16
NCHUNK = N // L
G = 4
NC = 2
NS = 16
NW = NC * NS
TPP = NW // SCP
RPT = N // TPP
RCHUNKS = RPT // L


def _tc_body(x_ref, y_ref, o_ref, xa, ya):
    s = pl.program_id(0)

    @pl.when(s == 0)
    def _():
        xa[4:5, :] = jnp.ones((1, N), jnp.float32)
        xa[5:8, :] = jnp.zeros((3, N), jnp.float32)
        ya[3:4, :] = jnp.ones((1, N), jnp.float32)
        ya[5:8, :] = jnp.zeros((3, N), jnp.float32)
        o_ref[0, 0] = jnp.float32(0.0)

    acc = jnp.float32(0.0)
    for q in range(P):
        xb = x_ref[q]
        yb = y_ref[q]
        x2 = jnp.sum(xb * xb, axis=0)
        y2 = jnp.sum(yb * yb, axis=0)
        xa[0:3, :] = xb * -2.0
        xa[3:4, :] = x2[None, :]
        ya[0:3, :] = yb
        ya[4:5, :] = y2[None, :]
        xav = xa[...]
        yav = ya[...]
        runmin = None
        colsum = jnp.float32(0.0)
        for c in range(N // CW):
            yc = yav[:, c * CW:(c + 1) * CW]
            dc = lax.dot_general(xav, yc, (((0,), (0,)), ((), ())),
                                 preferred_element_type=jnp.float32)
            runmin = dc if c == 0 else jnp.minimum(runmin, dc)
            colsum = colsum + jnp.sum(jnp.min(dc, axis=0))
        rowsum = jnp.sum(jnp.min(runmin, axis=1))
        pid = s * P + q
        w = jnp.where(pid % 8 == 0, jnp.float32(2.0), jnp.float32(1.0))
        acc = acc + w * (rowsum + colsum)

    o_ref[0, 0] += acc


_tc_call = pl.pallas_call(
    _tc_body,
    grid=(KTC // P,),
    in_specs=[
        pl.BlockSpec((P, 3, N), lambda s: (s, 0, 0)),
        pl.BlockSpec((P, 3, N), lambda s: (s, 0, 0)),
    ],
    out_specs=pl.BlockSpec(memory_space=pltpu.SMEM),
    out_shape=jax.ShapeDtypeStruct((1, 1), jnp.float32),
    scratch_shapes=[
        pltpu.VMEM((KA, N), jnp.float32),
        pltpu.VMEM((KA, N), jnp.float32),
    ],
    compiler_params=pltpu.CompilerParams(
        dimension_semantics=("arbitrary",),
    ),
)


_mesh = plsc.VectorSubcoreMesh(core_axis_name="c", subcore_axis_name="s")


@functools.partial(
    pl.kernel,
    mesh=_mesh,
    out_type=(
        jax.ShapeDtypeStruct((NW, L), jnp.float32),
        jax.ShapeDtypeStruct((NW, N), jnp.float32),
    ),
    scratch_types=[
        pltpu.VMEM((RPT,), jnp.float32),
        pltpu.VMEM((RPT,), jnp.float32),
        pltpu.VMEM((RPT,), jnp.float32),
        pltpu.VMEM((RPT,), jnp.float32),
        pltpu.VMEM((N,), jnp.float32),
        pltpu.VMEM((N,), jnp.float32),
        pltpu.VMEM((N,), jnp.float32),
        pltpu.VMEM((N,), jnp.float32),
        pltpu.VMEM((N,), jnp.float32),
        pltpu.VMEM((L,), jnp.float32),
    ],
)
def _chamfer_sc(x_hbm, y_hbm, row_out, col_out,
                xs0, xs1, xs2, xsq, ym0, ym1, ym2, ysq, colbuf, accbuf):
    wid = lax.axis_index("s") * NC + lax.axis_index("c")
    u = wid // TPP
    quarter = wid % TPP
    r0 = quarter * RPT
    inf = jnp.full((L,), jnp.inf, jnp.float32)
    lane_iota = lax.iota(jnp.int32, L)

    def lane_min(v):
        m = v
        for sh in (8, 4, 2, 1):
            m = jnp.minimum(m, m.at[lane_iota ^ sh].get(mode="promise_in_bounds"))
        return m

    pltpu.sync_copy(x_hbm.at[u * 3 + 0, pl.ds(r0, RPT)], xs0)
    pltpu.sync_copy(x_hbm.at[u * 3 + 1, pl.ds(r0, RPT)], xs1)
    pltpu.sync_copy(x_hbm.at[u * 3 + 2, pl.ds(r0, RPT)], xs2)
    pltpu.sync_copy(y_hbm.at[u * 3 + 0], ym0)
    pltpu.sync_copy(y_hbm.at[u * 3 + 1], ym1)
    pltpu.sync_copy(y_hbm.at[u * 3 + 2], ym2)

    def prep_y(c, _):
        s = pl.ds(c * L, L)
        a0 = ym0[s]
        a1 = ym1[s]
        a2 = ym2[s]
        ysq[s] = a0 * a0 + a1 * a1 + a2 * a2
        ym0[s] = a0 * -2.0
        ym1[s] = a1 * -2.0
        ym2[s] = a2 * -2.0
        colbuf[s] = inf
        return 0

    lax.fori_loop(0, NCHUNK, prep_y, 0)

    def prep_x(c, _):
        s = pl.ds(c * L, L)
        b0 = xs0[s]
        b1 = xs1[s]
        b2 = xs2[s]
        xsq[s] = b0 * b0 + b1 * b1 + b2 * b2
        return 0

    lax.fori_loop(0, RCHUNKS, prep_x, 0)

    def rowchunk(rc, rowacc):
        rs = pl.ds(rc * L, L)
        xv0 = xs0[rs]
        xv1 = xs1[rs]
        xv2 = xs2[rs]
        xvq = xsq[rs]
        for sb in range(L // G):
            xb = []
            for g in range(G):
                lane = sb * G + g
                xb.append((jnp.full((L,), xv0[lane]),
                           jnp.full((L,), xv1[lane]),
                           jnp.full((L,), xv2[lane]),
                           jnp.full((L,), xvq[lane])))

            def chunk(c, rows):
                s = pl.ds(c * L, L)
                m0 = ym0[s]
                m1 = ym1[s]
                m2 = ym2[s]
                ys = ysq[s]
                cm = colbuf[s]
                new_rows = []
                for g in range(G):
                    e = (ys + xb[g][3]) + xb[g][0] * m0 + xb[g][1] * m1 + xb[g][2] * m2
                    new_rows.append(jnp.minimum(rows[g], e))
                    cm = jnp.minimum(cm, e)
                colbuf[s] = cm
                return tuple(new_rows)

            rows = lax.fori_loop(0, NCHUNK, chunk, (inf,) * G)
            for g in range(G):
                rowacc = rowacc + lane_min(rows[g])
        return rowacc

    rowacc = lax.fori_loop(0, RCHUNKS, rowchunk, jnp.zeros((L,), jnp.float32))

    accbuf[...] = rowacc * 0.0625
    pltpu.sync_copy(accbuf, row_out.at[wid])
    pltpu.sync_copy(colbuf, col_out.at[wid])


def kernel(X_v, target_X_v):
    x = jnp.transpose(X_v.reshape(NPAIR, N, 3), (0, 2, 1))
    y = jnp.transpose(target_X_v.reshape(NPAIR, N, 3), (0, 2, 1))
    row_out, col_out = _chamfer_sc(x[KTC:].reshape(SCP * 3, N),
                                   y[KTC:].reshape(SCP * 3, N))
    tc_out = _tc_call(x[:KTC], y[:KTC])
    colmin = jnp.min(col_out.reshape(SCP, TPP, N), axis=1)
    pair_sums = (jnp.sum(row_out.reshape(SCP, TPP * L), axis=1)
                 + jnp.sum(colmin, axis=1))
    wts = jnp.where(jnp.arange(KTC, NPAIR) % 8 == 0, 2.0, 1.0)
    sc_total = jnp.sum(wts * pair_sums)
    return (tc_out[0, 0] + sc_total) * 0.125


# speedup vs baseline: 1.0600x; 1.0600x over previous
import functools

import jax
import jax.numpy as jnp
from jax import lax
from jax.experimental import pallas as pl
from jax.experimental.pallas import tpu as pltpu
from jax.experimental.pallas import tpu_sc as plsc

N = 1024
NPAIR = 64
CW = 128
KA = 8
P = 8
KTC = 56
SCP = NPAIR - KTC
L = 16
NCHUNK = N // L
G = 4
NC = 2
NS = 16
NW = NC * NS
TPP = NW // SCP
RPT = N // TPP
RCHUNKS = RPT // L


def _tc_body(x_ref, y_ref, o_ref, xa, ya):
    s = pl.program_id(0)

    @pl.when(s == 0)
    def _():
        xa[4:5, :] = jnp.ones((1, N), jnp.float32)
        xa[5:8, :] = jnp.zeros((3, N), jnp.float32)
        ya[3:4, :] = jnp.ones((1, N), jnp.float32)
        ya[5:8, :] = jnp.zeros((3, N), jnp.float32)
        o_ref[0, 0] = jnp.float32(0.0)

    acc = jnp.float32(0.0)
    for q in range(P):
        xb = x_ref[q]
        yb = y_ref[q]
        x2 = jnp.sum(xb * xb, axis=0)
        y2 = jnp.sum(yb * yb, axis=0)
        xa[0:3, :] = xb * -2.0
        xa[3:4, :] = x2[None, :]
        ya[0:3, :] = yb
        ya[4:5, :] = y2[None, :]
        xav = xa[...]
        yav = ya[...]
        runmin = None
        colsum = jnp.float32(0.0)
        for c in range(N // CW):
            yc = yav[:, c * CW:(c + 1) * CW]
            dc = lax.dot_general(xav, yc, (((0,), (0,)), ((), ())),
                                 preferred_element_type=jnp.float32)
            runmin = dc if c == 0 else jnp.minimum(runmin, dc)
            colsum = colsum + jnp.sum(jnp.min(dc, axis=0))
        rowsum = jnp.sum(jnp.min(runmin, axis=1))
        pid = s * P + q
        w = jnp.where(pid % 8 == 0, jnp.float32(2.0), jnp.float32(1.0))
        acc = acc + w * (rowsum + colsum)

    o_ref[0, 0] += acc


_tc_call = pl.pallas_call(
    _tc_body,
    grid=(KTC // P,),
    in_specs=[
        pl.BlockSpec((P, 3, N), lambda s: (s, 0, 0)),
        pl.BlockSpec((P, 3, N), lambda s: (s, 0, 0)),
    ],
    out_specs=pl.BlockSpec(memory_space=pltpu.SMEM),
    out_shape=jax.ShapeDtypeStruct((1, 1), jnp.float32),
    scratch_shapes=[
        pltpu.VMEM((KA, N), jnp.float32),
        pltpu.VMEM((KA, N), jnp.float32),
    ],
    compiler_params=pltpu.CompilerParams(
        dimension_semantics=("arbitrary",),
    ),
)


_mesh = plsc.VectorSubcoreMesh(core_axis_name="c", subcore_axis_name="s")


@functools.partial(
    pl.kernel,
    mesh=_mesh,
    out_type=(
        jax.ShapeDtypeStruct((NW, L), jnp.float32),
        jax.ShapeDtypeStruct((NW, N), jnp.float32),
    ),
    scratch_types=[
        pltpu.VMEM((N,), jnp.float32),
        pltpu.VMEM((N,), jnp.float32),
        pltpu.VMEM((N,), jnp.float32),
        pltpu.VMEM((N,), jnp.float32),
        pltpu.VMEM((N,), jnp.float32),
        pltpu.VMEM((N,), jnp.float32),
        pltpu.VMEM((N,), jnp.float32),
        pltpu.VMEM((N,), jnp.float32),
        pltpu.VMEM((N,), jnp.float32),
        pltpu.VMEM((L,), jnp.float32),
    ],
)
def _chamfer_sc(x_hbm, y_hbm, row_out, col_out,
                xs0, xs1, xs2, xsq, ym0, ym1, ym2, ysq, colbuf, accbuf):
    wid = lax.axis_index("s") * NC + lax.axis_index("c")
    u = wid // TPP
    p = KTC + u
    quarter = wid % TPP
    r0 = quarter * RPT
    inf = jnp.full((L,), jnp.inf, jnp.float32)
    lane_iota = lax.iota(jnp.int32, L)

    def lane_min(v):
        m = v
        for sh in (8, 4, 2, 1):
            m = jnp.minimum(m, m.at[lane_iota ^ sh].get(mode="promise_in_bounds"))
        return m

    pltpu.sync_copy(x_hbm.at[p * 3 + 0], xs0)
    pltpu.sync_copy(x_hbm.at[p * 3 + 1], xs1)
    pltpu.sync_copy(x_hbm.at[p * 3 + 2], xs2)
    pltpu.sync_copy(y_hbm.at[p * 3 + 0], ym0)
    pltpu.sync_copy(y_hbm.at[p * 3 + 1], ym1)
    pltpu.sync_copy(y_hbm.at[p * 3 + 2], ym2)

    def prep_y(c, _):
        s = pl.ds(c * L, L)
        a0 = ym0[s]
        a1 = ym1[s]
        a2 = ym2[s]
        ysq[s] = a0 * a0 + a1 * a1 + a2 * a2
        ym0[s] = a0 * -2.0
        ym1[s] = a1 * -2.0
        ym2[s] = a2 * -2.0
        colbuf[s] = inf
        return 0

    lax.fori_loop(0, NCHUNK, prep_y, 0)

    def prep_x(c, _):
        s = pl.ds(r0 + c * L, L)
        b0 = xs0[s]
        b1 = xs1[s]
        b2 = xs2[s]
        xsq[s] = b0 * b0 + b1 * b1 + b2 * b2
        return 0

    lax.fori_loop(0, RCHUNKS, prep_x, 0)

    def rowchunk(rc, rowacc):
        rs = pl.ds(r0 + rc * L, L)
        xv0 = xs0[rs]
        xv1 = xs1[rs]
        xv2 = xs2[rs]
        xvq = xsq[rs]
        for sb in range(L // G):
            xb = []
            for g in range(G):
                lane = sb * G + g
                xb.append((jnp.full((L,), xv0[lane]),
                           jnp.full((L,), xv1[lane]),
                           jnp.full((L,), xv2[lane]),
                           jnp.full((L,), xvq[lane])))

            def chunk(c, rows):
                s = pl.ds(c * L, L)
                m0 = ym0[s]
                m1 = ym1[s]
                m2 = ym2[s]
                ys = ysq[s]
                cm = colbuf[s]
                new_rows = []
                for g in range(G):
                    e = (ys + xb[g][3]) + xb[g][0] * m0 + xb[g][1] * m1 + xb[g][2] * m2
                    new_rows.append(jnp.minimum(rows[g], e))
                    cm = jnp.minimum(cm, e)
                colbuf[s] = cm
                return tuple(new_rows)

            rows = lax.fori_loop(0, NCHUNK, chunk, (inf,) * G)
            for g in range(G):
                rowacc = rowacc + lane_min(rows[g])
        return rowacc

    rowacc = lax.fori_loop(0, RCHUNKS, rowchunk, jnp.zeros((L,), jnp.float32))

    accbuf[...] = rowacc * 0.0625
    pltpu.sync_copy(accbuf, row_out.at[wid])
    pltpu.sync_copy(colbuf, col_out.at[wid])


def kernel(X_v, target_X_v):
    x = jnp.transpose(X_v.reshape(NPAIR, N, 3), (0, 2, 1))
    y = jnp.transpose(target_X_v.reshape(NPAIR, N, 3), (0, 2, 1))
    row_out, col_out = _chamfer_sc(x.reshape(NPAIR * 3, N),
                                   y.reshape(NPAIR * 3, N))
    tc_out = _tc_call(x, y)
    colmin = jnp.min(col_out.reshape(SCP, TPP, N), axis=1)
    pair_sums = (jnp.sum(row_out.reshape(SCP, TPP * L), axis=1)
                 + jnp.sum(colmin, axis=1))
    wts = jnp.where(jnp.arange(KTC, NPAIR) % 8 == 0, 2.0, 1.0)
    sc_total = jnp.sum(wts * pair_sums)
    return (tc_out[0, 0] + sc_total) * 0.125


# speedup vs baseline: 1.0673x; 1.0069x over previous
import functools

import jax
import jax.numpy as jnp
from jax import lax
from jax.experimental import pallas as pl
from jax.experimental.pallas import tpu as pltpu
from jax.experimental.pallas import tpu_sc as plsc

N = 1024
NPAIR = 64
CW = 128
KA = 8
P = 8
KTC = 56
SCP = NPAIR - KTC
L = 16
NCHUNK = N // L
G = 4
NC = 2
NS = 16
NW = NC * NS
TPP = NW // SCP
RPT = N // TPP
RCHUNKS = RPT // L


def _tc_body(x_ref, y_ref, o_ref, xa, ya):
    s = pl.program_id(0)

    @pl.when(s == 0)
    def _():
        xa[4:5, :] = jnp.ones((1, N), jnp.float32)
        xa[5:8, :] = jnp.zeros((3, N), jnp.float32)
        ya[3:4, :] = jnp.ones((1, N), jnp.float32)
        ya[5:8, :] = jnp.zeros((3, N), jnp.float32)
        o_ref[0, 0] = jnp.float32(0.0)

    acc = jnp.float32(0.0)
    x3 = x_ref[...]
    y3 = y_ref[...]
    for q in range(P):
        xb = x3[3 * q:3 * q + 3]
        yb = y3[3 * q:3 * q + 3]
        x2 = jnp.sum(xb * xb, axis=0)
        y2 = jnp.sum(yb * yb, axis=0)
        xa[0:3, :] = xb * -2.0
        xa[3:4, :] = x2[None, :]
        ya[0:3, :] = yb
        ya[4:5, :] = y2[None, :]
        xav = xa[...]
        yav = ya[...]
        runmin = None
        colsum = jnp.float32(0.0)
        for c in range(N // CW):
            yc = yav[:, c * CW:(c + 1) * CW]
            dc = lax.dot_general(xav, yc, (((0,), (0,)), ((), ())),
                                 preferred_element_type=jnp.float32)
            runmin = dc if c == 0 else jnp.minimum(runmin, dc)
            colsum = colsum + jnp.sum(jnp.min(dc, axis=0))
        rowsum = jnp.sum(jnp.min(runmin, axis=1))
        pid = s * P + q
        w = jnp.where(pid % 8 == 0, jnp.float32(2.0), jnp.float32(1.0))
        acc = acc + w * (rowsum + colsum)

    o_ref[0, 0] += acc


_tc_call = pl.pallas_call(
    _tc_body,
    grid=(KTC // P,),
    in_specs=[
        pl.BlockSpec((P * 3, N), lambda s: (s, 0)),
        pl.BlockSpec((P * 3, N), lambda s: (s, 0)),
    ],
    out_specs=pl.BlockSpec(memory_space=pltpu.SMEM),
    out_shape=jax.ShapeDtypeStruct((1, 1), jnp.float32),
    scratch_shapes=[
        pltpu.VMEM((KA, N), jnp.float32),
        pltpu.VMEM((KA, N), jnp.float32),
    ],
    compiler_params=pltpu.CompilerParams(
        dimension_semantics=("arbitrary",),
    ),
)


_mesh = plsc.VectorSubcoreMesh(core_axis_name="c", subcore_axis_name="s")


@functools.partial(
    pl.kernel,
    mesh=_mesh,
    out_type=(
        jax.ShapeDtypeStruct((NW, L), jnp.float32),
        jax.ShapeDtypeStruct((NW, N), jnp.float32),
    ),
    scratch_types=[
        pltpu.VMEM((N,), jnp.float32),
        pltpu.VMEM((N,), jnp.float32),
        pltpu.VMEM((N,), jnp.float32),
        pltpu.VMEM((N,), jnp.float32),
        pltpu.VMEM((N,), jnp.float32),
        pltpu.VMEM((N,), jnp.float32),
        pltpu.VMEM((N,), jnp.float32),
        pltpu.VMEM((N,), jnp.float32),
        pltpu.VMEM((N,), jnp.float32),
        pltpu.VMEM((L,), jnp.float32),
    ],
)
def _chamfer_sc(x_hbm, y_hbm, row_out, col_out,
                xs0, xs1, xs2, xsq, ym0, ym1, ym2, ysq, colbuf, accbuf):
    wid = lax.axis_index("s") * NC + lax.axis_index("c")
    u = wid // TPP
    p = KTC + u
    quarter = wid % TPP
    r0 = quarter * RPT
    inf = jnp.full((L,), jnp.inf, jnp.float32)
    lane_iota = lax.iota(jnp.int32, L)

    def lane_min(v):
        m = v
        for sh in (8, 4, 2, 1):
            m = jnp.minimum(m, m.at[lane_iota ^ sh].get(mode="promise_in_bounds"))
        return m

    pltpu.sync_copy(x_hbm.at[p * 3 + 0], xs0)
    pltpu.sync_copy(x_hbm.at[p * 3 + 1], xs1)
    pltpu.sync_copy(x_hbm.at[p * 3 + 2], xs2)
    pltpu.sync_copy(y_hbm.at[p * 3 + 0], ym0)
    pltpu.sync_copy(y_hbm.at[p * 3 + 1], ym1)
    pltpu.sync_copy(y_hbm.at[p * 3 + 2], ym2)

    def prep_y(c, _):
        s = pl.ds(c * L, L)
        a0 = ym0[s]
        a1 = ym1[s]
        a2 = ym2[s]
        ysq[s] = a0 * a0 + a1 * a1 + a2 * a2
        ym0[s] = a0 * -2.0
        ym1[s] = a1 * -2.0
        ym2[s] = a2 * -2.0
        colbuf[s] = inf
        return 0

    lax.fori_loop(0, NCHUNK, prep_y, 0)

    def prep_x(c, _):
        s = pl.ds(r0 + c * L, L)
        b0 = xs0[s]
        b1 = xs1[s]
        b2 = xs2[s]
        xsq[s] = b0 * b0 + b1 * b1 + b2 * b2
        return 0

    lax.fori_loop(0, RCHUNKS, prep_x, 0)

    def rowchunk(rc, rowacc):
        rs = pl.ds(r0 + rc * L, L)
        xv0 = xs0[rs]
        xv1 = xs1[rs]
        xv2 = xs2[rs]
        xvq = xsq[rs]
        for sb in range(L // G):
            xb = []
            for g in range(G):
                lane = sb * G + g
                xb.append((jnp.full((L,), xv0[lane]),
                           jnp.full((L,), xv1[lane]),
                           jnp.full((L,), xv2[lane]),
                           jnp.full((L,), xvq[lane])))

            def chunk(c, rows):
                s = pl.ds(c * L, L)
                m0 = ym0[s]
                m1 = ym1[s]
                m2 = ym2[s]
                ys = ysq[s]
                cm = colbuf[s]
                new_rows = []
                for g in range(G):
                    e = (ys + xb[g][3]) + xb[g][0] * m0 + xb[g][1] * m1 + xb[g][2] * m2
                    new_rows.append(jnp.minimum(rows[g], e))
                    cm = jnp.minimum(cm, e)
                colbuf[s] = cm
                return tuple(new_rows)

            rows = lax.fori_loop(0, NCHUNK, chunk, (inf,) * G)
            for g in range(G):
                rowacc = rowacc + lane_min(rows[g])
        return rowacc

    rowacc = lax.fori_loop(0, RCHUNKS, rowchunk, jnp.zeros((L,), jnp.float32))

    accbuf[...] = rowacc * 0.0625
    pltpu.sync_copy(accbuf, row_out.at[wid])
    pltpu.sync_copy(colbuf, col_out.at[wid])


def kernel(X_v, target_X_v):
    x = jnp.transpose(X_v.reshape(NPAIR, N, 3), (0, 2, 1)).reshape(NPAIR * 3, N)
    y = jnp.transpose(target_X_v.reshape(NPAIR, N, 3), (0, 2, 1)).reshape(NPAIR * 3, N)
    row_out, col_out = _chamfer_sc(x, y)
    tc_out = _tc_call(x, y)
    colmin = jnp.min(col_out.reshape(SCP, TPP, N), axis=1)
    pair_sums = (jnp.sum(row_out.reshape(SCP, TPP * L), axis=1)
                 + jnp.sum(colmin, axis=1))
    wts = jnp.where(jnp.arange(KTC, NPAIR) % 8 == 0, 2.0, 1.0)
    sc_total = jnp.sum(wts * pair_sums)
    return (tc_out[0, 0] + sc_total) * 0.125


# speedup vs baseline: 1.3685x; 1.2822x over previous
import functools

import jax
import jax.numpy as jnp
from jax import lax
from jax.experimental import pallas as pl
from jax.experimental.pallas import tpu as pltpu
from jax.experimental.pallas import tpu_sc as plsc

N = 1024
NPAIR = 64
CW = 128
KA = 8
P = 8
KTC = 64
SCP = 8
L = 16
NCHUNK = N // L
G = 4
NC = 2
NS = 16
NW = NC * NS
TPP = NW // SCP
RPT = N // TPP
RCHUNKS = RPT // L


def _tc_body(x_ref, y_ref, o_ref, xa, ya):
    s = pl.program_id(0)

    @pl.when(s == 0)
    def _():
        xa[4:5, :] = jnp.ones((1, N), jnp.float32)
        xa[5:8, :] = jnp.zeros((3, N), jnp.float32)
        ya[3:4, :] = jnp.ones((1, N), jnp.float32)
        ya[5:8, :] = jnp.zeros((3, N), jnp.float32)
        o_ref[0, 0] = jnp.float32(0.0)

    acc = jnp.float32(0.0)
    x3 = x_ref[...]
    y3 = y_ref[...]
    for q in range(P):
        xb = x3[3 * q:3 * q + 3]
        yb = y3[3 * q:3 * q + 3]
        x2 = jnp.sum(xb * xb, axis=0)
        y2 = jnp.sum(yb * yb, axis=0)
        xa[0:3, :] = xb * -2.0
        xa[3:4, :] = x2[None, :]
        ya[0:3, :] = yb
        ya[4:5, :] = y2[None, :]
        xav = xa[...]
        yav = ya[...]
        runmin = None
        colsum = jnp.float32(0.0)
        for c in range(N // CW):
            yc = yav[:, c * CW:(c + 1) * CW]
            dc = lax.dot_general(xav, yc, (((0,), (0,)), ((), ())),
                                 preferred_element_type=jnp.float32)
            runmin = dc if c == 0 else jnp.minimum(runmin, dc)
            colsum = colsum + jnp.sum(jnp.min(dc, axis=0))
        rowsum = jnp.sum(jnp.min(runmin, axis=1))
        pid = s * P + q
        w = jnp.where(pid % 8 == 0, jnp.float32(2.0), jnp.float32(1.0))
        acc = acc + w * (rowsum + colsum)

    o_ref[0, 0] += acc


_tc_call = pl.pallas_call(
    _tc_body,
    grid=(KTC // P,),
    in_specs=[
        pl.BlockSpec((P * 3, N), lambda s: (s, 0)),
        pl.BlockSpec((P * 3, N), lambda s: (s, 0)),
    ],
    out_specs=pl.BlockSpec(memory_space=pltpu.SMEM),
    out_shape=jax.ShapeDtypeStruct((1, 1), jnp.float32),
    scratch_shapes=[
        pltpu.VMEM((KA, N), jnp.float32),
        pltpu.VMEM((KA, N), jnp.float32),
    ],
    compiler_params=pltpu.CompilerParams(
        dimension_semantics=("arbitrary",),
    ),
)


_mesh = plsc.VectorSubcoreMesh(core_axis_name="c", subcore_axis_name="s")


@functools.partial(
    pl.kernel,
    mesh=_mesh,
    out_type=(
        jax.ShapeDtypeStruct((NW, L), jnp.float32),
        jax.ShapeDtypeStruct((NW, N), jnp.float32),
    ),
    scratch_types=[
        pltpu.VMEM((N,), jnp.float32),
        pltpu.VMEM((N,), jnp.float32),
        pltpu.VMEM((N,), jnp.float32),
        pltpu.VMEM((N,), jnp.float32),
        pltpu.VMEM((N,), jnp.float32),
        pltpu.VMEM((N,), jnp.float32),
        pltpu.VMEM((N,), jnp.float32),
        pltpu.VMEM((N,), jnp.float32),
        pltpu.VMEM((N,), jnp.float32),
        pltpu.VMEM((L,), jnp.float32),
    ],
)
def _chamfer_sc(x_hbm, y_hbm, row_out, col_out,
                xs0, xs1, xs2, xsq, ym0, ym1, ym2, ysq, colbuf, accbuf):
    wid = lax.axis_index("s") * NC + lax.axis_index("c")
    u = wid // TPP
    p = KTC + u
    quarter = wid % TPP
    r0 = quarter * RPT
    inf = jnp.full((L,), jnp.inf, jnp.float32)
    lane_iota = lax.iota(jnp.int32, L)

    def lane_min(v):
        m = v
        for sh in (8, 4, 2, 1):
            m = jnp.minimum(m, m.at[lane_iota ^ sh].get(mode="promise_in_bounds"))
        return m

    pltpu.sync_copy(x_hbm.at[p * 3 + 0], xs0)
    pltpu.sync_copy(x_hbm.at[p * 3 + 1], xs1)
    pltpu.sync_copy(x_hbm.at[p * 3 + 2], xs2)
    pltpu.sync_copy(y_hbm.at[p * 3 + 0], ym0)
    pltpu.sync_copy(y_hbm.at[p * 3 + 1], ym1)
    pltpu.sync_copy(y_hbm.at[p * 3 + 2], ym2)

    def prep_y(c, _):
        s = pl.ds(c * L, L)
        a0 = ym0[s]
        a1 = ym1[s]
        a2 = ym2[s]
        ysq[s] = a0 * a0 + a1 * a1 + a2 * a2
        ym0[s] = a0 * -2.0
        ym1[s] = a1 * -2.0
        ym2[s] = a2 * -2.0
        colbuf[s] = inf
        return 0

    lax.fori_loop(0, NCHUNK, prep_y, 0)

    def prep_x(c, _):
        s = pl.ds(r0 + c * L, L)
        b0 = xs0[s]
        b1 = xs1[s]
        b2 = xs2[s]
        xsq[s] = b0 * b0 + b1 * b1 + b2 * b2
        return 0

    lax.fori_loop(0, RCHUNKS, prep_x, 0)

    def rowchunk(rc, rowacc):
        rs = pl.ds(r0 + rc * L, L)
        xv0 = xs0[rs]
        xv1 = xs1[rs]
        xv2 = xs2[rs]
        xvq = xsq[rs]
        for sb in range(L // G):
            xb = []
            for g in range(G):
                lane = sb * G + g
                xb.append((jnp.full((L,), xv0[lane]),
                           jnp.full((L,), xv1[lane]),
                           jnp.full((L,), xv2[lane]),
                           jnp.full((L,), xvq[lane])))

            def chunk(c, rows):
                s = pl.ds(c * L, L)
                m0 = ym0[s]
                m1 = ym1[s]
                m2 = ym2[s]
                ys = ysq[s]
                cm = colbuf[s]
                new_rows = []
                for g in range(G):
                    e = (ys + xb[g][3]) + xb[g][0] * m0 + xb[g][1] * m1 + xb[g][2] * m2
                    new_rows.append(jnp.minimum(rows[g], e))
                    cm = jnp.minimum(cm, e)
                colbuf[s] = cm
                return tuple(new_rows)

            rows = lax.fori_loop(0, NCHUNK, chunk, (inf,) * G)
            for g in range(G):
                rowacc = rowacc + lane_min(rows[g])
        return rowacc

    rowacc = lax.fori_loop(0, RCHUNKS, rowchunk, jnp.zeros((L,), jnp.float32))

    accbuf[...] = rowacc * 0.0625
    pltpu.sync_copy(accbuf, row_out.at[wid])
    pltpu.sync_copy(colbuf, col_out.at[wid])


def kernel(X_v, target_X_v):
    x = jnp.transpose(X_v.reshape(NPAIR, N, 3), (0, 2, 1)).reshape(NPAIR * 3, N)
    y = jnp.transpose(target_X_v.reshape(NPAIR, N, 3), (0, 2, 1)).reshape(NPAIR * 3, N)
    tc_out = _tc_call(x, y)
    return tc_out[0, 0] * 0.125
